# baseline (speedup 1.0000x reference)
import jax
import jax.numpy as jnp
from jax.experimental import pallas as pl
from jax.experimental.pallas import tpu as pltpu

_CHUNK_ROWS = 5000
_DEPTH = 4


def kernel(embed_user, embed_item):
    n, d = embed_user.shape
    chunk = _CHUNK_ROWS if n % _CHUNK_ROWS == 0 else n
    nchunks = n // chunk
    total = 2 * nchunks
    depth = min(_DEPTH, total)

    def body(user_hbm, item_hbm, out_hbm, buf, *sems):
        sem_in, sem_out = sems[:depth], sems[depth:]
        srcs = (user_hbm, item_hbm)

        def mk(k):
            t, c = divmod(k, nchunks)
            p = k % depth
            lo = c * chunk
            load = pltpu.make_async_copy(
                srcs[t].at[pl.ds(lo, chunk)], buf.at[p], sem_in[p])
            store = pltpu.make_async_copy(
                buf.at[p], out_hbm.at[t, pl.ds(lo, chunk)], sem_out[p])
            return load, store

        ops = [mk(k) for k in range(total)]
        for k in range(depth):
            ops[k][0].start()
        for k in range(total):
            if k >= 1 and k - 1 + depth < total:
                ops[k - 1][1].wait()
                ops[k - 1 + depth][0].start()
            ops[k][0].wait()
            ops[k][1].start()
        for k in range(max(0, total - depth), total):
            ops[k][1].wait()

    return pl.pallas_call(
        body,
        out_shape=jax.ShapeDtypeStruct((2, n, d), embed_user.dtype),
        in_specs=[
            pl.BlockSpec(memory_space=pltpu.MemorySpace.HBM),
            pl.BlockSpec(memory_space=pltpu.MemorySpace.HBM),
        ],
        out_specs=pl.BlockSpec(memory_space=pltpu.MemorySpace.HBM),
        scratch_shapes=(
            [pltpu.VMEM((depth, chunk, d), embed_user.dtype)]
            + [pltpu.SemaphoreType.DMA] * (2 * depth)
        ),
    )(embed_user, embed_item)
